# baseline (speedup 1.0000x reference)
import functools

import jax
import jax.numpy as jnp
from jax import lax
from jax.experimental import pallas as pl
from jax.experimental.pallas import tpu as pltpu
from jax.experimental.pallas import tpu_sc as plsc

D_MODEL = 256
SEQ = 200
VOCAB = 6
BATCH = 1024
EPS = 1e-5

_NC = 2
_NS = 16
_NW = _NC * _NS

N_TOK = BATCH * SEQ
GCH = 80
CHUNK = 2 * GCH
NBUF = 2
N_CHUNKS = N_TOK // CHUNK
CHUNKS_PER_W = N_CHUNKS // _NW
ROWS_PER_W = CHUNKS_PER_W * CHUNK


NREP = 32


def _table_idx_kernel(x_ref, tok_ref, pos_ref, gamma_ref, beta_ref,
                      table_ref, idx_ref):
    emb = tok_ref[...][:, None, :] + pos_ref[...][None, :, :]
    mean = jnp.mean(emb, axis=-1, keepdims=True)
    cent = emb - mean
    var = jnp.mean(cent * cent, axis=-1, keepdims=True)
    normed = cent * lax.rsqrt(var + EPS)
    full = (normed * gamma_ref[...][None, :, :]
            + beta_ref[...][None, :, :])
    table_ref[...] = jnp.broadcast_to(full[None], table_ref.shape)
    s_iota = lax.broadcasted_iota(jnp.int32, idx_ref.shape, 1)
    b_iota = lax.broadcasted_iota(jnp.int32, idx_ref.shape, 0)
    rep = lax.rem(lax.div(b_iota, BATCH // _NW), NREP)
    idx_ref[...] = x_ref[...] * SEQ + s_iota + rep * (VOCAB * SEQ)


@functools.lru_cache(maxsize=1)
def _make_sc_gather():
    mesh = plsc.VectorSubcoreMesh(core_axis_name="c", subcore_axis_name="s")

    @functools.partial(
        pl.kernel,
        mesh=mesh,
        out_type=jax.ShapeDtypeStruct((N_TOK, D_MODEL), jnp.float32),
        scratch_types=[
            pltpu.VMEM((2 * CHUNKS_PER_W, GCH), jnp.int32),
            pltpu.VMEM((NBUF, CHUNK, D_MODEL), jnp.float32),
        ]
        + [pltpu.SemaphoreType.DMA] * (2 * NBUF),
    )
    def _sc_gather(table_hbm, idx_hbm, out_hbm, idx_v, rows, *sems):
        gsems = sems[:NBUF]
        osems = sems[NBUF:]
        wid = lax.axis_index("s") * _NC + lax.axis_index("c")
        row_base = wid * ROWS_PER_W
        pltpu.sync_copy(idx_hbm.at[wid], idx_v)

        def gath(i, b):
            pltpu.async_copy(table_hbm.at[idx_v.at[2 * i]],
                             rows.at[b, pl.ds(0, GCH)], gsems[b])
            pltpu.async_copy(table_hbm.at[idx_v.at[2 * i + 1]],
                             rows.at[b, pl.ds(GCH, GCH)], gsems[b])

        def gath_wait(i, b):
            pltpu.make_async_copy(
                table_hbm.at[idx_v.at[2 * i]],
                rows.at[b, pl.ds(0, GCH)], gsems[b]).wait()
            pltpu.make_async_copy(
                table_hbm.at[idx_v.at[2 * i + 1]],
                rows.at[b, pl.ds(GCH, GCH)], gsems[b]).wait()

        def scat(i, b):
            pltpu.async_copy(
                rows.at[b],
                out_hbm.at[pl.ds(row_base + i * CHUNK, CHUNK)], osems[b])

        def scat_wait(i, b):
            pltpu.make_async_copy(
                rows.at[b],
                out_hbm.at[pl.ds(row_base + i * CHUNK, CHUNK)],
                osems[b]).wait()

        gath(0, 0)
        gath_wait(0, 0)
        scat(0, 0)
        gath(1, 1)

        def body(g, carry):
            i1 = 2 * g + 1
            gath_wait(i1, 1)
            scat(i1, 1)
            scat_wait(i1 - 1, 0)
            gath(i1 + 1, 0)
            i2 = 2 * g + 2
            gath_wait(i2, 0)
            scat(i2, 0)
            scat_wait(i2 - 1, 1)
            gath(i2 + 1, 1)
            return carry

        lax.fori_loop(0, (CHUNKS_PER_W - 2) // 2, body, 0)
        last = CHUNKS_PER_W - 1
        gath_wait(last, 1)
        scat(last, 1)
        scat_wait(last - 1, 0)
        scat_wait(last, 1)

    return _sc_gather


def kernel(x, tok_embed, pos_embed, gamma, beta):
    x = x.astype(jnp.int32)
    table3, idx2 = pl.pallas_call(
        _table_idx_kernel,
        out_shape=(
            jax.ShapeDtypeStruct((NREP, VOCAB, SEQ, D_MODEL), jnp.float32),
            jax.ShapeDtypeStruct((BATCH, SEQ), jnp.int32),
        ),
    )(x, tok_embed, pos_embed,
      gamma.reshape(1, D_MODEL), beta.reshape(1, D_MODEL))

    table = table3.reshape(NREP * VOCAB * SEQ, D_MODEL)
    idx = idx2.reshape(_NW, 2 * CHUNKS_PER_W, GCH)
    out = _make_sc_gather()(table, idx)
    return out.reshape(BATCH, SEQ, D_MODEL)


# speedup vs baseline: 1.0404x; 1.0404x over previous
import functools

import jax
import jax.numpy as jnp
from jax import lax
from jax.experimental import pallas as pl
from jax.experimental.pallas import tpu as pltpu
from jax.experimental.pallas import tpu_sc as plsc

D_MODEL = 256
SEQ = 200
VOCAB = 6
BATCH = 1024
EPS = 1e-5

_NC = 2
_NS = 16
_NW = _NC * _NS

N_TOK = BATCH * SEQ
GCH = 80
CHUNK = 2 * GCH
NBUF = 2
N_CHUNKS = N_TOK // CHUNK
CHUNKS_PER_W = N_CHUNKS // _NW
ROWS_PER_W = CHUNKS_PER_W * CHUNK


NREP = 16


def _table_idx_kernel(x_ref, tok_ref, pos_ref, gamma_ref, beta_ref,
                      table_ref, idx_ref):
    emb = tok_ref[...][:, None, :] + pos_ref[...][None, :, :]
    mean = jnp.mean(emb, axis=-1, keepdims=True)
    cent = emb - mean
    var = jnp.mean(cent * cent, axis=-1, keepdims=True)
    normed = cent * lax.rsqrt(var + EPS)
    full = (normed * gamma_ref[...][None, :, :]
            + beta_ref[...][None, :, :])
    table_ref[...] = jnp.broadcast_to(full[None], table_ref.shape)
    s_iota = lax.broadcasted_iota(jnp.int32, idx_ref.shape, 1)
    b_iota = lax.broadcasted_iota(jnp.int32, idx_ref.shape, 0)
    rep = lax.rem(lax.div(b_iota, BATCH // _NW), NREP)
    idx_ref[...] = x_ref[...] * SEQ + s_iota + rep * (VOCAB * SEQ)


@functools.lru_cache(maxsize=1)
def _make_sc_gather():
    mesh = plsc.VectorSubcoreMesh(core_axis_name="c", subcore_axis_name="s")

    @functools.partial(
        pl.kernel,
        mesh=mesh,
        out_type=jax.ShapeDtypeStruct((N_TOK, D_MODEL), jnp.float32),
        scratch_types=[
            pltpu.VMEM((2 * CHUNKS_PER_W, GCH), jnp.int32),
            pltpu.VMEM((NBUF, CHUNK, D_MODEL), jnp.float32),
        ]
        + [pltpu.SemaphoreType.DMA] * (2 * NBUF),
    )
    def _sc_gather(table_hbm, idx_hbm, out_hbm, idx_v, rows, *sems):
        gsems = sems[:NBUF]
        osems = sems[NBUF:]
        wid = lax.axis_index("s") * _NC + lax.axis_index("c")
        row_base = wid * ROWS_PER_W
        pltpu.sync_copy(idx_hbm.at[wid], idx_v)

        def gath(i, b):
            pltpu.async_copy(table_hbm.at[idx_v.at[2 * i]],
                             rows.at[b, pl.ds(0, GCH)], gsems[b])
            pltpu.async_copy(table_hbm.at[idx_v.at[2 * i + 1]],
                             rows.at[b, pl.ds(GCH, GCH)], gsems[b])

        def gath_wait(i, b):
            pltpu.make_async_copy(
                table_hbm.at[idx_v.at[2 * i]],
                rows.at[b, pl.ds(0, GCH)], gsems[b]).wait()
            pltpu.make_async_copy(
                table_hbm.at[idx_v.at[2 * i + 1]],
                rows.at[b, pl.ds(GCH, GCH)], gsems[b]).wait()

        def scat(i, b):
            pltpu.async_copy(
                rows.at[b],
                out_hbm.at[pl.ds(row_base + i * CHUNK, CHUNK)], osems[b])

        def scat_wait(i, b):
            pltpu.make_async_copy(
                rows.at[b],
                out_hbm.at[pl.ds(row_base + i * CHUNK, CHUNK)],
                osems[b]).wait()

        gath(0, 0)
        gath_wait(0, 0)
        scat(0, 0)
        gath(1, 1)

        def body(g, carry):
            i1 = 2 * g + 1
            gath_wait(i1, 1)
            scat(i1, 1)
            scat_wait(i1 - 1, 0)
            gath(i1 + 1, 0)
            i2 = 2 * g + 2
            gath_wait(i2, 0)
            scat(i2, 0)
            scat_wait(i2 - 1, 1)
            gath(i2 + 1, 1)
            return carry

        lax.fori_loop(0, (CHUNKS_PER_W - 2) // 2, body, 0)
        last = CHUNKS_PER_W - 1
        gath_wait(last, 1)
        scat(last, 1)
        scat_wait(last - 1, 0)
        scat_wait(last, 1)

    return _sc_gather


def kernel(x, tok_embed, pos_embed, gamma, beta):
    x = x.astype(jnp.int32)
    table3, idx2 = pl.pallas_call(
        _table_idx_kernel,
        out_shape=(
            jax.ShapeDtypeStruct((NREP, VOCAB, SEQ, D_MODEL), jnp.float32),
            jax.ShapeDtypeStruct((BATCH, SEQ), jnp.int32),
        ),
    )(x, tok_embed, pos_embed,
      gamma.reshape(1, D_MODEL), beta.reshape(1, D_MODEL))

    table = table3.reshape(NREP * VOCAB * SEQ, D_MODEL)
    idx = idx2.reshape(_NW, 2 * CHUNKS_PER_W, GCH)
    out = _make_sc_gather()(table, idx)
    return out.reshape(BATCH, SEQ, D_MODEL)


# speedup vs baseline: 1.0455x; 1.0050x over previous
import functools

import jax
import jax.numpy as jnp
from jax import lax
from jax.experimental import pallas as pl
from jax.experimental.pallas import tpu as pltpu
from jax.experimental.pallas import tpu_sc as plsc

D_MODEL = 256
SEQ = 200
VOCAB = 6
BATCH = 1024
EPS = 1e-5

_NC = 2
_NS = 16
_NW = _NC * _NS

N_TOK = BATCH * SEQ
GCH = 80
CHUNK = 2 * GCH
NBUF = 2
N_CHUNKS = N_TOK // CHUNK
CHUNKS_PER_W = N_CHUNKS // _NW
ROWS_PER_W = CHUNKS_PER_W * CHUNK


NREP = 8


def _table_idx_kernel(x_ref, tok_ref, pos_ref, gamma_ref, beta_ref,
                      table_ref, idx_ref):
    emb = tok_ref[...][:, None, :] + pos_ref[...][None, :, :]
    mean = jnp.mean(emb, axis=-1, keepdims=True)
    cent = emb - mean
    var = jnp.mean(cent * cent, axis=-1, keepdims=True)
    normed = cent * lax.rsqrt(var + EPS)
    full = (normed * gamma_ref[...][None, :, :]
            + beta_ref[...][None, :, :])
    table_ref[...] = jnp.broadcast_to(full[None], table_ref.shape)
    s_iota = lax.broadcasted_iota(jnp.int32, idx_ref.shape, 1)
    b_iota = lax.broadcasted_iota(jnp.int32, idx_ref.shape, 0)
    rep = lax.rem(lax.div(b_iota, BATCH // _NW), NREP)
    idx_ref[...] = x_ref[...] * SEQ + s_iota + rep * (VOCAB * SEQ)


@functools.lru_cache(maxsize=1)
def _make_sc_gather():
    mesh = plsc.VectorSubcoreMesh(core_axis_name="c", subcore_axis_name="s")

    @functools.partial(
        pl.kernel,
        mesh=mesh,
        out_type=jax.ShapeDtypeStruct((N_TOK, D_MODEL), jnp.float32),
        scratch_types=[
            pltpu.VMEM((2 * CHUNKS_PER_W, GCH), jnp.int32),
            pltpu.VMEM((NBUF, CHUNK, D_MODEL), jnp.float32),
        ]
        + [pltpu.SemaphoreType.DMA] * (2 * NBUF),
    )
    def _sc_gather(table_hbm, idx_hbm, out_hbm, idx_v, rows, *sems):
        gsems = sems[:NBUF]
        osems = sems[NBUF:]
        wid = lax.axis_index("s") * _NC + lax.axis_index("c")
        row_base = wid * ROWS_PER_W
        pltpu.sync_copy(idx_hbm.at[wid], idx_v)

        def gath(i, b):
            pltpu.async_copy(table_hbm.at[idx_v.at[2 * i]],
                             rows.at[b, pl.ds(0, GCH)], gsems[b])
            pltpu.async_copy(table_hbm.at[idx_v.at[2 * i + 1]],
                             rows.at[b, pl.ds(GCH, GCH)], gsems[b])

        def gath_wait(i, b):
            pltpu.make_async_copy(
                table_hbm.at[idx_v.at[2 * i]],
                rows.at[b, pl.ds(0, GCH)], gsems[b]).wait()
            pltpu.make_async_copy(
                table_hbm.at[idx_v.at[2 * i + 1]],
                rows.at[b, pl.ds(GCH, GCH)], gsems[b]).wait()

        def scat(i, b):
            pltpu.async_copy(
                rows.at[b],
                out_hbm.at[pl.ds(row_base + i * CHUNK, CHUNK)], osems[b])

        def scat_wait(i, b):
            pltpu.make_async_copy(
                rows.at[b],
                out_hbm.at[pl.ds(row_base + i * CHUNK, CHUNK)],
                osems[b]).wait()

        gath(0, 0)
        gath_wait(0, 0)
        scat(0, 0)
        gath(1, 1)

        def body(g, carry):
            i1 = 2 * g + 1
            gath_wait(i1, 1)
            scat(i1, 1)
            scat_wait(i1 - 1, 0)
            gath(i1 + 1, 0)
            i2 = 2 * g + 2
            gath_wait(i2, 0)
            scat(i2, 0)
            scat_wait(i2 - 1, 1)
            gath(i2 + 1, 1)
            return carry

        lax.fori_loop(0, (CHUNKS_PER_W - 2) // 2, body, 0)
        last = CHUNKS_PER_W - 1
        gath_wait(last, 1)
        scat(last, 1)
        scat_wait(last - 1, 0)
        scat_wait(last, 1)

    return _sc_gather


def kernel(x, tok_embed, pos_embed, gamma, beta):
    x = x.astype(jnp.int32)
    table3, idx2 = pl.pallas_call(
        _table_idx_kernel,
        out_shape=(
            jax.ShapeDtypeStruct((NREP, VOCAB, SEQ, D_MODEL), jnp.float32),
            jax.ShapeDtypeStruct((BATCH, SEQ), jnp.int32),
        ),
    )(x, tok_embed, pos_embed,
      gamma.reshape(1, D_MODEL), beta.reshape(1, D_MODEL))

    table = table3.reshape(NREP * VOCAB * SEQ, D_MODEL)
    idx = idx2.reshape(_NW, 2 * CHUNKS_PER_W, GCH)
    out = _make_sc_gather()(table, idx)
    return out.reshape(BATCH, SEQ, D_MODEL)


# speedup vs baseline: 1.0488x; 1.0031x over previous
import functools

import jax
import jax.numpy as jnp
from jax import lax
from jax.experimental import pallas as pl
from jax.experimental.pallas import tpu as pltpu
from jax.experimental.pallas import tpu_sc as plsc

D_MODEL = 256
SEQ = 200
VOCAB = 6
BATCH = 1024
EPS = 1e-5

_NC = 2
_NS = 16
_NW = _NC * _NS

N_TOK = BATCH * SEQ
GCH = 80
CHUNK = 2 * GCH
NBUF = 2
N_CHUNKS = N_TOK // CHUNK
CHUNKS_PER_W = N_CHUNKS // _NW
ROWS_PER_W = CHUNKS_PER_W * CHUNK


NREP = 4


def _table_idx_kernel(x_ref, tok_ref, pos_ref, gamma_ref, beta_ref,
                      table_ref, idx_ref):
    emb = tok_ref[...][:, None, :] + pos_ref[...][None, :, :]
    mean = jnp.mean(emb, axis=-1, keepdims=True)
    cent = emb - mean
    var = jnp.mean(cent * cent, axis=-1, keepdims=True)
    normed = cent * lax.rsqrt(var + EPS)
    full = (normed * gamma_ref[...][None, :, :]
            + beta_ref[...][None, :, :])
    table_ref[...] = jnp.broadcast_to(full[None], table_ref.shape)
    s_iota = lax.broadcasted_iota(jnp.int32, idx_ref.shape, 1)
    b_iota = lax.broadcasted_iota(jnp.int32, idx_ref.shape, 0)
    rep = lax.rem(lax.div(b_iota, BATCH // _NW), NREP)
    idx_ref[...] = x_ref[...] * SEQ + s_iota + rep * (VOCAB * SEQ)


@functools.lru_cache(maxsize=1)
def _make_sc_gather():
    mesh = plsc.VectorSubcoreMesh(core_axis_name="c", subcore_axis_name="s")

    @functools.partial(
        pl.kernel,
        mesh=mesh,
        out_type=jax.ShapeDtypeStruct((N_TOK, D_MODEL), jnp.float32),
        scratch_types=[
            pltpu.VMEM((2 * CHUNKS_PER_W, GCH), jnp.int32),
            pltpu.VMEM((NBUF, CHUNK, D_MODEL), jnp.float32),
        ]
        + [pltpu.SemaphoreType.DMA] * (2 * NBUF),
    )
    def _sc_gather(table_hbm, idx_hbm, out_hbm, idx_v, rows, *sems):
        gsems = sems[:NBUF]
        osems = sems[NBUF:]
        wid = lax.axis_index("s") * _NC + lax.axis_index("c")
        row_base = wid * ROWS_PER_W
        pltpu.sync_copy(idx_hbm.at[wid], idx_v)

        def gath(i, b):
            pltpu.async_copy(table_hbm.at[idx_v.at[2 * i]],
                             rows.at[b, pl.ds(0, GCH)], gsems[b])
            pltpu.async_copy(table_hbm.at[idx_v.at[2 * i + 1]],
                             rows.at[b, pl.ds(GCH, GCH)], gsems[b])

        def gath_wait(i, b):
            pltpu.make_async_copy(
                table_hbm.at[idx_v.at[2 * i]],
                rows.at[b, pl.ds(0, GCH)], gsems[b]).wait()
            pltpu.make_async_copy(
                table_hbm.at[idx_v.at[2 * i + 1]],
                rows.at[b, pl.ds(GCH, GCH)], gsems[b]).wait()

        def scat(i, b):
            pltpu.async_copy(
                rows.at[b],
                out_hbm.at[pl.ds(row_base + i * CHUNK, CHUNK)], osems[b])

        def scat_wait(i, b):
            pltpu.make_async_copy(
                rows.at[b],
                out_hbm.at[pl.ds(row_base + i * CHUNK, CHUNK)],
                osems[b]).wait()

        gath(0, 0)
        gath_wait(0, 0)
        scat(0, 0)
        gath(1, 1)

        def body(g, carry):
            i1 = 2 * g + 1
            gath_wait(i1, 1)
            scat(i1, 1)
            scat_wait(i1 - 1, 0)
            gath(i1 + 1, 0)
            i2 = 2 * g + 2
            gath_wait(i2, 0)
            scat(i2, 0)
            scat_wait(i2 - 1, 1)
            gath(i2 + 1, 1)
            return carry

        lax.fori_loop(0, (CHUNKS_PER_W - 2) // 2, body, 0)
        last = CHUNKS_PER_W - 1
        gath_wait(last, 1)
        scat(last, 1)
        scat_wait(last - 1, 0)
        scat_wait(last, 1)

    return _sc_gather


def kernel(x, tok_embed, pos_embed, gamma, beta):
    x = x.astype(jnp.int32)
    table3, idx2 = pl.pallas_call(
        _table_idx_kernel,
        out_shape=(
            jax.ShapeDtypeStruct((NREP, VOCAB, SEQ, D_MODEL), jnp.float32),
            jax.ShapeDtypeStruct((BATCH, SEQ), jnp.int32),
        ),
    )(x, tok_embed, pos_embed,
      gamma.reshape(1, D_MODEL), beta.reshape(1, D_MODEL))

    table = table3.reshape(NREP * VOCAB * SEQ, D_MODEL)
    idx = idx2.reshape(_NW, 2 * CHUNKS_PER_W, GCH)
    out = _make_sc_gather()(table, idx)
    return out.reshape(BATCH, SEQ, D_MODEL)
